# Initial kernel scaffold; baseline (speedup 1.0000x reference)
import jax
import jax.numpy as jnp
from jax.experimental import pallas as pl


def kernel(edge_index, adj_values, feature, W1, b1, W2, b2, Wlin, blin):
    raise NotImplementedError("write your pallas kernel here")


# speedup vs baseline: 7.7039x; 7.7039x over previous
import functools

import jax
import jax.numpy as jnp
from jax import lax
from jax.experimental import pallas as pl
from jax.experimental.pallas import tpu as pltpu
import jax.experimental.pallas.tpu_sc as plsc

NC = 2
NS = 16
L = 16
NW = NC * NS

CHUNK = 128

_MESH = dict(core_axis_name="c", subcore_axis_name="s", num_cores=NC,
             num_subcores=NS)


def _tc_support1(feature, W1):
    def body(f_ref, w_ref, o_ref):
        o_ref[...] = jnp.dot(f_ref[...], w_ref[...],
                             preferred_element_type=jnp.float32)
    return pl.pallas_call(
        body,
        out_shape=jax.ShapeDtypeStruct((feature.shape[0], W1.shape[1]),
                                       jnp.float32),
    )(feature, W1)


def _tc_middle(parts, b1, W2, Wlin):
    def body(p_ref, b1_ref, w2_ref, wl_ref, v_ref):
        h = jax.nn.relu(p_ref[0] + p_ref[1] + b1_ref[...][None, :])
        w2l = jnp.dot(w2_ref[...], wl_ref[...],
                      preferred_element_type=jnp.float32)
        v_ref[...] = jnp.dot(h, w2l, preferred_element_type=jnp.float32)
    n_pad = parts.shape[1]
    return pl.pallas_call(
        body,
        out_shape=jax.ShapeDtypeStruct((n_pad, 1), jnp.float32),
    )(parts, b1, W2, Wlin)


def _tc_final(q, b2, Wlin, blin, n):
    def body(q_ref, b2_ref, wl_ref, bl_ref, o_ref):
        c = jnp.dot(b2_ref[...][None, :], wl_ref[...],
                    preferred_element_type=jnp.float32)[0, 0] + bl_ref[0]
        s = q_ref[0, :n] + q_ref[1, :n] + c
        o_ref[...] = jax.nn.sigmoid(s)[:, None]
    return pl.pallas_call(
        body,
        out_shape=jax.ShapeDtypeStruct((n, 1), jnp.float32),
    )(q, b2, Wlin, blin)


def _sc_spmm_wide(src3, dst3, adj3, sup, n_pad, d1, nchunk):
    rows_per_tile = n_pad // NS
    assert rows_per_tile % CHUNK == 0

    @functools.partial(
        pl.kernel,
        out_type=jax.ShapeDtypeStruct((NC * n_pad, d1), jnp.float32),
        mesh=plsc.VectorSubcoreMesh(**_MESH),
        compiler_params=pltpu.CompilerParams(use_tc_tiling_on_sc=False),
        scratch_types=[
            pltpu.VMEM((nchunk, CHUNK), jnp.int32),
            pltpu.VMEM((nchunk, CHUNK), jnp.int32),
            pltpu.VMEM((nchunk, CHUNK), jnp.float32),
            pltpu.VMEM((CHUNK, d1), jnp.float32),
            pltpu.VMEM_SHARED((n_pad, d1), jnp.float32),
            pltpu.SemaphoreType.DMA,
        ],
    )
    def spmm1(src_hbm, dst_hbm, adj_hbm, sup_hbm, out_hbm,
              src_v, dst_v, adj_v, buf, acc, sem):
        c = lax.axis_index("c")
        s = lax.axis_index("s")
        wid = s * NC + c

        @pl.loop(0, CHUNK)
        def _zrow(r):
            for j in range(d1 // L):
                buf[r, pl.ds(j * L, L)] = jnp.zeros((L,), jnp.float32)

        @pl.loop(0, rows_per_tile // CHUNK)
        def _zacc(i):
            pltpu.sync_copy(
                buf, acc.at[pl.ds(s * rows_per_tile + i * CHUNK, CHUNK)])
        plsc.subcore_barrier()

        pltpu.sync_copy(src_hbm.at[wid], src_v)
        pltpu.sync_copy(dst_hbm.at[wid], dst_v)
        pltpu.sync_copy(adj_hbm.at[wid], adj_v)

        @pl.loop(0, nchunk)
        def _edges(ch):
            pltpu.async_copy(sup_hbm.at[src_v.at[ch]], buf, sem).wait()

            @pl.loop(0, CHUNK // L)
            def _scale(k):
                a16 = adj_v[ch, pl.ds(k * L, L)]
                for r2 in range(L):
                    av = jnp.full((L,), a16[r2])
                    row = k * L + r2
                    for j in range(d1 // L):
                        buf[row, pl.ds(j * L, L)] = (
                            buf[row, pl.ds(j * L, L)] * av)

            pltpu.sync_copy(buf, acc.at[dst_v.at[ch]], add=True)
        plsc.subcore_barrier()

        @pl.loop(0, rows_per_tile // CHUNK)
        def _out(i):
            base = s * rows_per_tile + i * CHUNK
            pltpu.sync_copy(acc.at[pl.ds(base, CHUNK)], buf)
            pltpu.sync_copy(buf, out_hbm.at[pl.ds(c * n_pad + base, CHUNK)])

    out = spmm1(src3, dst3, adj3, sup)
    return out.reshape(NC, n_pad, d1)


def _sc_spmm_scalar(src3, dst3, adj3, v1d, n_pad, nchunk):
    rows_per_tile = n_pad // NS

    @functools.partial(
        pl.kernel,
        out_type=jax.ShapeDtypeStruct((NC, n_pad), jnp.float32),
        mesh=plsc.VectorSubcoreMesh(**_MESH),
        compiler_params=pltpu.CompilerParams(use_tc_tiling_on_sc=False,
                                             needs_layout_passes=False),
        scratch_types=[
            pltpu.VMEM((nchunk, CHUNK), jnp.int32),
            pltpu.VMEM((nchunk, CHUNK), jnp.int32),
            pltpu.VMEM((nchunk, CHUNK), jnp.float32),
            pltpu.VMEM((nchunk, CHUNK), jnp.float32),
            pltpu.VMEM((n_pad,), jnp.float32),
            pltpu.VMEM((rows_per_tile,), jnp.float32),
            pltpu.VMEM_SHARED((n_pad,), jnp.float32),
        ],
    )
    def spmm2(src_hbm, dst_hbm, adj_hbm, v_hbm, out_hbm,
              src_v, dst_v, adj_v, msg_v, vloc, obuf, acc):
        c = lax.axis_index("c")
        s = lax.axis_index("s")
        wid = s * NC + c

        @pl.loop(0, rows_per_tile // L)
        def _z(i):
            obuf[pl.ds(i * L, L)] = jnp.zeros((L,), jnp.float32)
        pltpu.sync_copy(obuf, acc.at[pl.ds(s * rows_per_tile,
                                           rows_per_tile)])
        plsc.subcore_barrier()

        pltpu.sync_copy(v_hbm, vloc)
        pltpu.sync_copy(src_hbm.at[wid], src_v)
        pltpu.sync_copy(dst_hbm.at[wid], dst_v)
        pltpu.sync_copy(adj_hbm.at[wid], adj_v)

        @pl.loop(0, nchunk)
        def _edges(ch):
            @pl.loop(0, CHUNK // L)
            def _msg(k):
                idx = src_v.at[ch][pl.ds(k * L, L)]
                vals = plsc.load_gather(vloc, [idx])
                msg_v.at[ch][pl.ds(k * L, L)] = (
                    vals * adj_v.at[ch][pl.ds(k * L, L)])
            pltpu.sync_copy(msg_v.at[ch], acc.at[dst_v.at[ch]], add=True)
        plsc.subcore_barrier()

        pltpu.sync_copy(acc.at[pl.ds(s * rows_per_tile, rows_per_tile)],
                        obuf)
        pltpu.sync_copy(obuf, out_hbm.at[c, pl.ds(s * rows_per_tile,
                                                  rows_per_tile)])

    return spmm2(src3, dst3, adj3, v1d)


def kernel(edge_index, adj_values, feature, W1, b1, W2, b2, Wlin, blin):
    n = feature.shape[0]
    e = edge_index.shape[1]
    d1 = W1.shape[1]

    rows_per_tile = -(-n // (NS * CHUNK)) * CHUNK
    n_pad = NS * rows_per_tile
    e_w = -(-e // (NW * CHUNK)) * CHUNK
    nchunk = e_w // CHUNK
    e_pad = NW * e_w

    src = edge_index[0].astype(jnp.int32)
    dst = edge_index[1].astype(jnp.int32)
    adv = adj_values.astype(jnp.float32)
    pad = e_pad - e
    if pad:
        src = jnp.concatenate([src, jnp.zeros((pad,), jnp.int32)])
        dst = jnp.concatenate([dst, jnp.zeros((pad,), jnp.int32)])
        adv = jnp.concatenate([adv, jnp.zeros((pad,), jnp.float32)])
    src3 = src.reshape(NW, nchunk, CHUNK)
    dst3 = dst.reshape(NW, nchunk, CHUNK)
    adj3 = adv.reshape(NW, nchunk, CHUNK)

    support1 = _tc_support1(feature, W1)
    parts = _sc_spmm_wide(src3, dst3, adj3, support1, n_pad, d1, nchunk)
    v = _tc_middle(parts, b1, W2, Wlin).reshape(n_pad)
    q = _sc_spmm_scalar(src3, dst3, adj3, v, n_pad, nchunk)
    return _tc_final(q, b2, Wlin, blin, n)
